# baseline (speedup 1.0000x reference)
import jax
import jax.numpy as jnp
from jax import lax
from jax.experimental import pallas as pl
from jax.experimental.pallas import tpu as pltpu

_D = 4


def _combine_body(t_ref, ac_ref, om_ref, x_hbm, n_hbm, o_hbm,
                  xb, nb, ob, xsem, nsem, osem):
    i = pl.program_id(0)
    nsteps = pl.num_programs(0)
    s = lax.rem(i, _D)

    @pl.when(i == 0)
    def _():
        for k in range(_D):
            pltpu.make_async_copy(x_hbm.at[k], xb.at[k], xsem.at[k]).start()
            pltpu.make_async_copy(n_hbm.at[k], nb.at[k], nsem.at[k]).start()

    pltpu.make_async_copy(x_hbm.at[i], xb.at[s], xsem.at[s]).wait()
    pltpu.make_async_copy(n_hbm.at[i], nb.at[s], nsem.at[s]).wait()

    @pl.when(i >= _D)
    def _():
        pltpu.make_async_copy(ob.at[s], o_hbm.at[i - _D], osem.at[s]).wait()

    tt = t_ref[i]
    c1 = ac_ref[tt]
    c2 = om_ref[tt]
    ob[s] = c1 * xb[s] + c2 * nb[s]

    pltpu.make_async_copy(ob.at[s], o_hbm.at[i], osem.at[s]).start()

    @pl.when(i + _D < nsteps)
    def _():
        pltpu.make_async_copy(x_hbm.at[i + _D], xb.at[s], xsem.at[s]).start()
        pltpu.make_async_copy(n_hbm.at[i + _D], nb.at[s], nsem.at[s]).start()

    @pl.when(i == nsteps - 1)
    def _():
        for k in range(_D):
            j = nsteps - _D + k
            pltpu.make_async_copy(ob.at[j % _D], o_hbm.at[j],
                                  osem.at[j % _D]).wait()


def kernel(x_start, t, noise, sqrt_alphas_cumprod, sqrt_one_minus_alphas_cumprod):
    B, C, H, W = x_start.shape

    smem = pl.BlockSpec(memory_space=pltpu.SMEM)
    hbm = pl.BlockSpec(memory_space=pltpu.MemorySpace.HBM)

    out = pl.pallas_call(
        _combine_body,
        grid=(B,),
        in_specs=[smem, smem, smem, hbm, hbm],
        out_specs=hbm,
        out_shape=jax.ShapeDtypeStruct((B, C, H, W), jnp.float32),
        scratch_shapes=[
            pltpu.VMEM((_D, C, H, W), jnp.float32),
            pltpu.VMEM((_D, C, H, W), jnp.float32),
            pltpu.VMEM((_D, C, H, W), jnp.float32),
            pltpu.SemaphoreType.DMA((_D,)),
            pltpu.SemaphoreType.DMA((_D,)),
            pltpu.SemaphoreType.DMA((_D,)),
        ],
    )(t.astype(jnp.int32), sqrt_alphas_cumprod, sqrt_one_minus_alphas_cumprod,
      x_start, noise)
    return out


# speedup vs baseline: 1.0103x; 1.0103x over previous
import jax
import jax.numpy as jnp
from jax.experimental import pallas as pl
from jax.experimental.pallas import tpu as pltpu


def _combine_body(t_ref, ac_ref, om_ref, x_ref, n_ref, o_ref):
    b = pl.program_id(0)
    tt = t_ref[b]
    c1 = ac_ref[tt]
    c2 = om_ref[tt]
    o_ref[...] = c1 * x_ref[...] + c2 * n_ref[...]


def kernel(x_start, t, noise, sqrt_alphas_cumprod, sqrt_one_minus_alphas_cumprod):
    B, C, H, W = x_start.shape

    smem = pl.BlockSpec(memory_space=pltpu.SMEM)
    blk = pl.BlockSpec((1, C, H, W), lambda b: (b, 0, 0, 0))

    out = pl.pallas_call(
        _combine_body,
        grid=(B,),
        in_specs=[smem, smem, smem, blk, blk],
        out_specs=blk,
        out_shape=jax.ShapeDtypeStruct((B, C, H, W), jnp.float32),
    )(t.astype(jnp.int32), sqrt_alphas_cumprod, sqrt_one_minus_alphas_cumprod,
      x_start, noise)
    return out
